# Initial kernel scaffold; baseline (speedup 1.0000x reference)
import jax
import jax.numpy as jnp
from jax.experimental import pallas as pl


def kernel(x, edge_index, W1, b1, W2, b2):
    raise NotImplementedError("write your pallas kernel here")


# speedup vs baseline: 16.6077x; 16.6077x over previous
import functools

import jax
import jax.numpy as jnp
from jax import lax
from jax.experimental import pallas as pl
from jax.experimental.pallas import tpu as pltpu
from jax.experimental.pallas import tpu_sc as plsc

N = 10000
E = 320000
D_IN = 128
H = 16
C = 2

NC = 2
NS = 16
NW = NC * NS
NPAD = 10112
CH = 80
EPT = 10080
NCH = EPT // CH
EPADT = EPT * NW
RPT = NPAD // NS


def _adj_body(t_hbm, src_hbm, dst_hbm, out_hbm,
              srcb, dstb, rows0, rows1, zbuf, acc, sem0, sem1):
    cid = lax.axis_index("c")
    sid = lax.axis_index("s")
    wid = cid * NS + sid

    def _zf(r, carry):
        zbuf[r] = jnp.zeros((16,), jnp.float32)
        return carry
    lax.fori_loop(0, RPT, _zf, 0)
    pltpu.sync_copy(zbuf, acc.at[pl.ds(sid * RPT, RPT)])

    pltpu.sync_copy(src_hbm.at[wid], srcb)
    pltpu.sync_copy(dst_hbm.at[wid], dstb)

    plsc.subcore_barrier()

    pltpu.async_copy(t_hbm.at[srcb.at[0]], rows0, sem0)
    pltpu.async_copy(t_hbm.at[srcb.at[1]], rows1, sem1)

    def _step(i, carry):
        c0 = 2 * i
        c1 = c0 + 1
        pltpu.make_async_copy(t_hbm.at[srcb.at[c0]], rows0, sem0).wait()
        pltpu.sync_copy(rows0, acc.at[dstb.at[c0]], add=True)
        pltpu.async_copy(t_hbm.at[srcb.at[c0 + 2]], rows0, sem0)
        pltpu.make_async_copy(t_hbm.at[srcb.at[c1]], rows1, sem1).wait()
        pltpu.sync_copy(rows1, acc.at[dstb.at[c1]], add=True)
        pltpu.async_copy(t_hbm.at[srcb.at[c1 + 2]], rows1, sem1)
        return carry
    lax.fori_loop(0, NCH // 2, _step, 0)

    pltpu.make_async_copy(t_hbm.at[srcb.at[NCH]], rows0, sem0).wait()
    pltpu.make_async_copy(t_hbm.at[srcb.at[NCH + 1]], rows1, sem1).wait()

    plsc.subcore_barrier()

    pltpu.sync_copy(acc.at[pl.ds(sid * RPT, RPT)],
                    out_hbm.at[pl.ds(cid * NPAD + sid * RPT, RPT)])


_adj_apply = pl.kernel(
    _adj_body,
    out_type=jax.ShapeDtypeStruct((2 * NPAD, 16), jnp.float32),
    mesh=plsc.VectorSubcoreMesh(core_axis_name="c", subcore_axis_name="s"),
    scratch_types=[
        pltpu.VMEM((NCH + 2, CH), jnp.int32),
        pltpu.VMEM((NCH + 2, CH), jnp.int32),
        pltpu.VMEM((CH, 16), jnp.float32),
        pltpu.VMEM((CH, 16), jnp.float32),
        pltpu.VMEM((RPT, 16), jnp.float32),
        pltpu.VMEM_SHARED((NPAD, 16), jnp.float32),
        pltpu.SemaphoreType.DMA,
        pltpu.SemaphoreType.DMA,
    ],
    compiler_params=pltpu.CompilerParams(use_tc_tiling_on_sc=False),
)


RB = 1264


def _prep1_body(dp0, dp1, x, w, b, t3, a1, a2, u0, dinv, d2):
    deg = dp0[...] + dp1[...]
    di = jnp.where(deg > 0, lax.rsqrt(jnp.maximum(deg, 1e-12)), 0.0)
    u = jnp.dot(x[...], w[...], preferred_element_type=jnp.float32)
    t3[...] = di * u[:, 48:64]
    a1[...] = di * u[:, 32:48]
    a2[...] = di * u[:, 16:32]
    u0[...] = u[:, 0:16] + b[...]
    dinv[...] = di
    d2[...] = di * di


def _mid_body(sp0, sp1, a, d2, t):
    t[...] = a[...] + d2[...] * (sp0[...] + sp1[...])


def _l2prep_body(sp0, sp1, u0, dinv, w, wp, t3, a1, a2, v0):
    h = u0[...] + dinv[...] * (sp0[...] + sp1[...])
    h = jnp.maximum(h, 0.0)
    vp = jnp.dot(h, wp[...], preferred_element_type=jnp.float32)
    t3[...] = dinv[...] * vp[:, 32:48]
    a1[...] = dinv[...] * vp[:, 16:32]
    a2[...] = dinv[...] * vp[:, 0:16]
    v0[...] = jnp.dot(h, w[...], preferred_element_type=jnp.float32)


def _final_body(sp0, sp1, v0, dinv, b, out):
    o = v0[...] + dinv[:, 0:2] * (sp0[:, 0:2] + sp1[:, 0:2]) + b[...]
    o0 = o[:, 0:1]
    o1 = o[:, 1:2]
    m = jnp.maximum(o0, o1)
    lse = m + jnp.log(jnp.exp(o0 - m) + jnp.exp(o1 - m))
    out[...] = o - lse


def _rows(width):
    return pl.BlockSpec((RB, width), lambda i: (i, 0))


def _full(r, c):
    return pl.BlockSpec((r, c), lambda i: (0, 0))


def _tc_call(body, in_specs, out_widths):
    return pl.pallas_call(
        body,
        grid=(NPAD // RB,),
        in_specs=in_specs,
        out_specs=[_rows(w) for w in out_widths],
        out_shape=[jax.ShapeDtypeStruct((NPAD, w), jnp.float32)
                   for w in out_widths],
    )


@jax.jit
def kernel(x, edge_index, W1, b1, W2, b2):
    f32 = jnp.float32

    x_p = jnp.zeros((NPAD, D_IN), f32).at[:N].set(x)
    pad = jnp.full((EPADT - E,), NPAD - 1, jnp.int32)
    zc = jnp.zeros((NW, 2, CH), jnp.int32)
    src3 = jnp.concatenate(
        [jnp.concatenate([edge_index[0], pad]).reshape(NW, NCH, CH), zc], 1)
    dst3 = jnp.concatenate(
        [jnp.concatenate([edge_index[1], pad]).reshape(NW, NCH, CH), zc], 1)

    w1r = jnp.concatenate([W1[k] for k in range(4)], axis=1)
    w2p = jnp.zeros((H, 48), f32)
    w2p = w2p.at[:, 0:2].set(W2[1]).at[:, 16:18].set(W2[2]) \
             .at[:, 32:34].set(W2[3])
    b1r = jnp.broadcast_to(b1[None, :], (NPAD, H)).astype(f32)
    b2r = jnp.broadcast_to(b2[None, :], (NPAD, C)).astype(f32)

    ones_t = jnp.ones((NPAD, 16), f32)

    dp = _adj_apply(ones_t, src3, dst3)
    prep1 = _tc_call(
        _prep1_body,
        [_rows(16), _rows(16), _rows(D_IN), _full(D_IN, 64), _rows(16)],
        [16] * 6)
    t3, a1, a2, u0, dinv, d2 = prep1(dp[:NPAD], dp[NPAD:], x_p, w1r, b1r)

    mid = _tc_call(_mid_body, [_rows(16)] * 4, [16])
    s = _adj_apply(t3, src3, dst3)
    t = mid(s[:NPAD], s[NPAD:], a1, d2)[0]
    s = _adj_apply(t, src3, dst3)
    t = mid(s[:NPAD], s[NPAD:], a2, d2)[0]
    s = _adj_apply(t, src3, dst3)

    l2prep = _tc_call(
        _l2prep_body,
        [_rows(16)] * 4 + [_full(H, C), _full(H, 48)],
        [16, 16, 16, C])
    t3b, a1b, a2b, v0 = l2prep(s[:NPAD], s[NPAD:], u0, dinv,
                               W2[0].astype(f32), w2p)

    s = _adj_apply(t3b, src3, dst3)
    t = mid(s[:NPAD], s[NPAD:], a1b, d2)[0]
    s = _adj_apply(t, src3, dst3)
    t = mid(s[:NPAD], s[NPAD:], a2b, d2)[0]
    s = _adj_apply(t, src3, dst3)

    fin = _tc_call(_final_body,
                   [_rows(16), _rows(16), _rows(C), _rows(16), _rows(C)],
                   [C])
    out = fin(s[:NPAD], s[NPAD:], v0, dinv, b2r)[0]
    return out[:N]
